# baseline (speedup 1.0000x reference)
import functools

import jax
import jax.numpy as jnp
from jax import lax
from jax.experimental import pallas as pl
from jax.experimental.pallas import tpu as pltpu
from jax.experimental.pallas import tpu_sc as plsc

B = 1024
F = 25
C = 16
D = 1536
EPS = 1e-5

_info = plsc.get_sparse_core_info()
NC = _info.num_cores
NS = _info.num_subcores
L = _info.num_lanes
NW = NC * NS

CAT_ROWS = F * B
ROWS_PER_W = CAT_ROWS // NW
BN_PER_W = B // NW

G = 16
NGROUPS = ROWS_PER_W // G


def _sc_body(idx_hbm, numx_hbm, table_hbm, emb_hbm, out_hbm,
             idx_v, bn_buf, emb_v, numx_v, d0, d1):
    sid = lax.axis_index("s")
    w = sid * NC + lax.axis_index("c")
    base = w * ROWS_PER_W

    pltpu.sync_copy(idx_hbm.at[w], idx_v)
    pltpu.sync_copy(emb_hbm, emb_v)
    pltpu.sync_copy(numx_hbm, numx_v)

    dsems = (d0, d1)

    def _drain_group(sem):
        pltpu.make_async_copy(table_hbm.at[pl.ds(0, G * D)],
                              out_hbm.at[pl.ds(0, G * D)], sem).wait()

    def group_body(g, _):
        off = g * G
        vec = idx_v[pl.ds(pl.multiple_of(off, 8), G)]

        for par in range(2):
            @pl.when((g >= 2) & (lax.rem(g, 2) == par))
            def _():
                _drain_group(dsems[par])

        for l in range(G):
            src = pl.multiple_of(vec[l] * D, 8)
            dst = pl.multiple_of((base + off + l) * D, 8)
            for par in range(2):
                @pl.when(lax.rem(g, 2) == par)
                def _():
                    pltpu.async_copy(table_hbm.at[pl.ds(src, D)],
                                     out_hbm.at[pl.ds(dst, D)], dsems[par])
        return 0

    lax.fori_loop(0, NGROUPS, group_body, 0)
    _drain_group(dsems[0])
    _drain_group(dsems[1])

    def stat_body(i, carry):
        s, sq = carry
        x = numx_v[pl.ds(i * L, L)]
        return s + x, sq + x * x

    zero = jnp.zeros((L,), jnp.float32)
    s, sq = lax.fori_loop(0, B // L, stat_body, (zero, zero))

    lanes = lax.iota(jnp.int32, L)
    _dnums = lax.GatherDimensionNumbers(
        offset_dims=(), collapsed_slice_dims=(0,), start_index_map=(0,))

    def _shuffle(x, idx):
        return lax.gather(x, idx[:, None], _dnums, (1,),
                          mode=lax.GatherScatterMode.PROMISE_IN_BOUNDS)

    def _splat_sum(x):
        for k in (8, 4, 2, 1):
            x = x + _shuffle(x, lanes ^ k)
        return x

    mv = _splat_sum(s) * (1.0 / B)
    ex2 = _splat_sum(sq) * (1.0 / B)
    vv = ex2 - mv * mv + EPS
    iv = plsc.bitcast(vv, jnp.int32)
    y = plsc.bitcast(jnp.full((L,), 0x5F3759DF, jnp.int32) - (iv >> 1),
                     jnp.float32)
    for _ in range(4):
        y = y * (1.5 - 0.5 * vv * y * y)

    half = BN_PER_W // 2
    for h in range(2):
        def row_body(i, _):
            bidx = w * BN_PER_W + h * half + i
            xb = plsc.load_gather(numx_v, [jnp.full((L,), bidx, jnp.int32)])
            xn = (xb - mv) * y
            for c in range(D // L):
                bn_buf[pl.ds(pl.multiple_of(i * D + c * L, 8), L)] = (
                    xn * emb_v[pl.ds(c * L, L)])
            return 0

        lax.fori_loop(0, half, row_body, 0)
        pltpu.sync_copy(
            bn_buf,
            out_hbm.at[pl.ds(
                pl.multiple_of((CAT_ROWS + w * BN_PER_W + h * half) * D, 8),
                half * D)])


@jax.jit
def _emb_layer(idx_flat, numx_flat, table_flat, num_emb):
    mesh = plsc.VectorSubcoreMesh(core_axis_name="c", subcore_axis_name="s")
    call = pl.kernel(
        _sc_body,
        out_type=jax.ShapeDtypeStruct(((F + 1) * B * D,), jnp.float32),
        mesh=mesh,
        scratch_types=[
            pltpu.VMEM((ROWS_PER_W,), jnp.int32),
            pltpu.VMEM(((BN_PER_W // 2) * D,), jnp.float32),
            pltpu.VMEM((D,), jnp.float32),
            pltpu.VMEM((B,), jnp.float32),
            pltpu.SemaphoreType.DMA,
            pltpu.SemaphoreType.DMA,
        ],
        compiler_params=pltpu.CompilerParams(needs_layout_passes=False),
    )
    return call(idx_flat, numx_flat, table_flat, num_emb)


def kernel(indices, num_x, tables, num_emb):
    idx = indices.astype(jnp.int32)
    idx_flat = (idx.T + (jnp.arange(F, dtype=jnp.int32) * C)[:, None])
    idx_flat = idx_flat.reshape(NW, ROWS_PER_W)
    table_flat = tables.reshape(F * C * D)
    numx_flat = num_x.reshape(B)
    out = _emb_layer(idx_flat, numx_flat, table_flat,
                     num_emb.astype(jnp.float32))
    return out.reshape((F + 1) * B, D)


# speedup vs baseline: 29.9504x; 29.9504x over previous
import functools

import jax
import jax.numpy as jnp
from jax import lax
from jax.experimental import pallas as pl
from jax.experimental.pallas import tpu as pltpu
from jax.experimental.pallas import tpu_sc as plsc

B = 1024
F = 25
C = 16
D = 1536
EPS = 1e-5

_info = plsc.get_sparse_core_info()
NC = _info.num_cores
NS = _info.num_subcores
L = _info.num_lanes
NW = NC * NS

CAT_ROWS = F * B
ROWS_PER_W = CAT_ROWS // NW
BN_PER_W = B // NW

G = 16
NCHUNKS = ROWS_PER_W // G
NBUF = 4


def _sc_body(idx_hbm, numx_hbm, table_hbm, emb_hbm, out_hbm,
             idx_v, buf, bn_buf, emb_v, numx_v, *sems):
    gsems = sems[:NBUF]
    ssems = sems[NBUF:]
    sid = lax.axis_index("s")
    w = sid * NC + lax.axis_index("c")
    base = w * ROWS_PER_W

    pltpu.sync_copy(idx_hbm.at[w], idx_v)
    pltpu.sync_copy(emb_hbm, emb_v)
    pltpu.sync_copy(numx_hbm, numx_v)

    def _idx_slice(m):
        return idx_v.at[pl.ds(pl.multiple_of(m * G, 8), G)]

    def _buf_at(p):
        return buf.at[pl.ds(pl.multiple_of(p * G, 8), G)]

    def _out_at(m):
        return out_hbm.at[pl.ds(pl.multiple_of(base + m * G, 8), G)]

    def _start_gather(m, p):
        pltpu.async_copy(table_hbm.at[_idx_slice(m)], _buf_at(p), gsems[p])

    def _wait_gather(m, p):
        pltpu.make_async_copy(table_hbm.at[_idx_slice(m)], _buf_at(p),
                              gsems[p]).wait()

    def _start_scatter(m, p):
        pltpu.async_copy(_buf_at(p), _out_at(m), ssems[p])

    def _wait_scatter(m, p):
        pltpu.make_async_copy(_buf_at(p), _out_at(m), ssems[p]).wait()

    for m in range(NBUF - 1):
        _start_gather(m, m)

    def chunk_body(m, _):
        for p in range(NBUF):
            @pl.when(lax.rem(m, NBUF) == p)
            def _():
                _wait_gather(m, p)
                _start_scatter(m, p)

            @pl.when((m + NBUF - 1 < NCHUNKS)
                     & (lax.rem(m + NBUF - 1, NBUF) == p))
            def _():
                @pl.when(m >= 1)
                def _():
                    _wait_scatter(m - 1, p)

                _start_gather(m + NBUF - 1, p)
        return 0

    lax.fori_loop(0, NCHUNKS, chunk_body, 0)
    for m in range(NCHUNKS - NBUF, NCHUNKS):
        _wait_scatter(m, m % NBUF)

    def stat_body(i, carry):
        s, sq = carry
        x = numx_v[pl.ds(i * L, L)]
        return s + x, sq + x * x

    zero = jnp.zeros((L,), jnp.float32)
    s, sq = lax.fori_loop(0, B // L, stat_body, (zero, zero))

    lanes = lax.iota(jnp.int32, L)
    _dnums = lax.GatherDimensionNumbers(
        offset_dims=(), collapsed_slice_dims=(0,), start_index_map=(0,))

    def _shuffle(x, idx):
        return lax.gather(x, idx[:, None], _dnums, (1,),
                          mode=lax.GatherScatterMode.PROMISE_IN_BOUNDS)

    def _splat_sum(x):
        for k in (8, 4, 2, 1):
            x = x + _shuffle(x, lanes ^ k)
        return x

    mv = _splat_sum(s) * (1.0 / B)
    ex2 = _splat_sum(sq) * (1.0 / B)
    vv = ex2 - mv * mv + EPS
    iv = plsc.bitcast(vv, jnp.int32)
    y = plsc.bitcast(jnp.full((L,), 0x5F3759DF, jnp.int32) - (iv >> 1),
                     jnp.float32)
    for _ in range(4):
        y = y * (1.5 - 0.5 * vv * y * y)

    half = BN_PER_W // 2
    for h in range(2):
        def row_body(i, _):
            bidx = w * BN_PER_W + h * half + i
            xb = plsc.load_gather(numx_v, [jnp.full((L,), bidx, jnp.int32)])
            xn = (xb - mv) * y
            for c in range(D // L):
                bn_buf[i, pl.ds(c * L, L)] = xn * emb_v[pl.ds(c * L, L)]
            return 0

        lax.fori_loop(0, half, row_body, 0)
        pltpu.sync_copy(
            bn_buf,
            out_hbm.at[pl.ds(CAT_ROWS + w * BN_PER_W + h * half, half)])


@jax.jit
def _emb_layer(idx_flat, numx_flat, table_flat, num_emb):
    mesh = plsc.VectorSubcoreMesh(core_axis_name="c", subcore_axis_name="s")
    call = pl.kernel(
        _sc_body,
        out_type=jax.ShapeDtypeStruct(((F + 1) * B, D), jnp.float32),
        mesh=mesh,
        scratch_types=[
            pltpu.VMEM((ROWS_PER_W,), jnp.int32),
            pltpu.VMEM((NBUF * G, D), jnp.float32),
            pltpu.VMEM((BN_PER_W // 2, D), jnp.float32),
            pltpu.VMEM((D,), jnp.float32),
            pltpu.VMEM((B,), jnp.float32),
        ] + [pltpu.SemaphoreType.DMA] * (2 * NBUF),
        compiler_params=pltpu.CompilerParams(needs_layout_passes=False),
    )
    return call(idx_flat, numx_flat, table_flat, num_emb)


def kernel(indices, num_x, tables, num_emb):
    idx = indices.astype(jnp.int32)
    idx_flat = (idx.T + (jnp.arange(F, dtype=jnp.int32) * C)[:, None])
    idx_flat = idx_flat.reshape(NW, ROWS_PER_W)
    table_flat = tables.reshape(F * C, D)
    numx_flat = num_x.reshape(B)
    return _emb_layer(idx_flat, numx_flat, table_flat,
                      num_emb.astype(jnp.float32))
